# Initial kernel scaffold; baseline (speedup 1.0000x reference)
import jax
import jax.numpy as jnp
from jax.experimental import pallas as pl


def kernel(x):
    raise NotImplementedError("write your pallas kernel here")


# speedup vs baseline: 2.0588x; 2.0588x over previous
import jax
import jax.numpy as jnp
from jax.experimental import pallas as pl

_STEP = 0.1
_OUTER = 15
_INNER = 40


def _tv2d_kernel(x_ref, o_ref):
    n = x_ref.shape[0]
    W = x_ref[...]

    col_ids = jax.lax.broadcasted_iota(jnp.int32, W.shape, 1)
    row_ids = jax.lax.broadcasted_iota(jnp.int32, W.shape, 0)
    mask_row = (col_ids < n - 1).astype(W.dtype)
    mask_col = (row_ids < n - 1).astype(W.dtype)

    def shl(a):
        return jnp.concatenate([a[:, 1:], jnp.zeros_like(a[:, :1])], axis=1)

    def shr(a):
        return jnp.concatenate([jnp.zeros_like(a[:, :1]), a[:, :-1]], axis=1)

    def shu(a):
        return jnp.concatenate([a[1:, :], jnp.zeros_like(a[:1, :])], axis=0)

    def shd(a):
        return jnp.concatenate([jnp.zeros_like(a[:1, :]), a[:-1, :]], axis=0)

    def prox(Y, axis):
        if axis == 1:
            fwd, bwd, mask = shl, shr, mask_row
        else:
            fwd, bwd, mask = shu, shd, mask_col

        z0 = jnp.zeros_like(Y)

        def body(_, carry):
            z, w, t = carry
            x = Y - (bwd(w) - w)
            z_new = jnp.clip(w + 0.25 * (fwd(x) - x), -_STEP, _STEP) * mask
            t_new = (1.0 + jnp.sqrt(1.0 + 4.0 * t * t)) / 2.0
            w_new = z_new + ((t - 1.0) / t_new) * (z_new - z)
            return (z_new, w_new, t_new)

        z, _, _ = jax.lax.fori_loop(
            0, _INNER, body, (z0, z0, jnp.float32(1.0)))
        return Y - (bwd(z) - z)

    def outer_body(_, carry):
        x, p, q = carry
        y = prox(x + p, axis=0)
        p = p + x - y
        x2 = prox(y + q, axis=1)
        q = q + y - x2
        return (x2, p, q)

    x, _, _ = jax.lax.fori_loop(
        0, _OUTER, outer_body, (W, jnp.zeros_like(W), jnp.zeros_like(W)))
    o_ref[...] = x


@jax.jit
def kernel(x):
    return pl.pallas_call(
        _tv2d_kernel,
        out_shape=jax.ShapeDtypeStruct(x.shape, x.dtype),
    )(x)
